# baseline (speedup 1.0000x reference)
import jax
import jax.numpy as jnp
from jax.experimental import pallas as pl
from jax.experimental.pallas import tpu as pltpu

_NC = 8


def _chamfer_tc_kernel(pred_ref, gt_ref, out_ref, s1_acc, s2_acc):
    b = pl.program_id(0)
    nb = pl.num_programs(0)

    p = pred_ref[0]
    gs = gt_ref[0] * -2.0
    N = p.shape[0]
    M = gs.shape[0]

    one_p = jnp.ones((N, 1), jnp.bfloat16)
    one_g = jnp.ones((M, 1), jnp.bfloat16)

    a2 = jnp.sum(p * p, axis=1, keepdims=True)
    a2_hi = a2.astype(jnp.bfloat16)
    a2_lo = (a2 - a2_hi.astype(jnp.float32)).astype(jnp.bfloat16)
    b2 = 0.25 * jnp.sum(gs * gs, axis=1, keepdims=True)
    b2_hi = b2.astype(jnp.bfloat16)
    b2_lo = (b2 - b2_hi.astype(jnp.float32)).astype(jnp.bfloat16)

    p_aug = jnp.concatenate(
        [p.astype(jnp.bfloat16), a2_hi, a2_lo, one_p, one_p], axis=1)
    g_aug = jnp.concatenate(
        [gs.astype(jnp.bfloat16), one_g, one_g, b2_hi, b2_lo], axis=1)

    @pl.when(b == 0)
    def _init_sums():
        s1_acc[...] = jnp.zeros_like(s1_acc)
        s2_acc[...] = jnp.zeros_like(s2_acc)

    cb = M // _NC
    rm_acc = None
    col_mins = []
    for c in range(_NC):
        lo, hi = c * cb, (c + 1) * cb
        e = jax.lax.dot_general(
            p_aug, g_aug[lo:hi, :],
            (((1,), (1,)), ((), ())),
            preferred_element_type=jnp.float32,
        )
        for k in range(cb // 128):
            ek = e[:, k * 128:(k + 1) * 128]
            rm_acc = ek if rm_acc is None else jnp.minimum(rm_acc, ek)
        col_mins.append(jnp.min(e, axis=0, keepdims=True))

    row_min = jnp.min(rm_acc, axis=1, keepdims=True)
    s1_acc[...] += jnp.sum(jnp.maximum(row_min, 0.0), keepdims=True)

    col_min = jnp.concatenate(col_mins, axis=1)
    s2_acc[...] += jnp.sum(jnp.maximum(col_min, 0.0), keepdims=True)

    @pl.when(b == nb - 1)
    def _emit():
        out_ref[...] = (s1_acc[...] / (nb * N)) + (s2_acc[...] / (nb * M))


def kernel(pred, gt):
    B, N, _ = pred.shape
    M = gt.shape[1]

    out = pl.pallas_call(
        _chamfer_tc_kernel,
        grid=(B,),
        in_specs=[
            pl.BlockSpec((1, N, 3), lambda b: (b, 0, 0)),
            pl.BlockSpec((1, M, 3), lambda b: (b, 0, 0)),
        ],
        out_specs=pl.BlockSpec((1, 1), lambda b: (0, 0)),
        out_shape=jax.ShapeDtypeStruct((1, 1), jnp.float32),
        scratch_shapes=[
            pltpu.VMEM((1, 1), jnp.float32),
            pltpu.VMEM((1, 1), jnp.float32),
        ],
    )(pred, gt)
    return out[0, 0]


# speedup vs baseline: 1.0342x; 1.0342x over previous
import jax
import jax.numpy as jnp
from jax.experimental import pallas as pl

_NC = 8


def _chamfer_tc_kernel(pred_ref, gs_ref, out_ref):
    B = pred_ref.shape[0]
    N = pred_ref.shape[1]
    M = gs_ref.shape[2]
    cb = M // _NC

    one_p = jnp.ones((N, 1), jnp.bfloat16)
    one_g = jnp.ones((1, M), jnp.bfloat16)

    s1 = jnp.zeros((1, 1), jnp.float32)
    s2 = jnp.zeros((1, 1), jnp.float32)
    for b in range(B):
        p = pred_ref[b]
        gs = gs_ref[b]

        a2 = jnp.sum(p * p, axis=1, keepdims=True)
        a2_hi = a2.astype(jnp.bfloat16)
        a2_lo = (a2 - a2_hi.astype(jnp.float32)).astype(jnp.bfloat16)
        b2 = 0.25 * jnp.sum(gs * gs, axis=0, keepdims=True)
        b2_hi = b2.astype(jnp.bfloat16)
        b2_lo = (b2 - b2_hi.astype(jnp.float32)).astype(jnp.bfloat16)

        p_aug = jnp.concatenate(
            [p.astype(jnp.bfloat16), a2_hi, a2_lo, one_p, one_p],
            axis=1)
        g_aug = jnp.concatenate(
            [gs.astype(jnp.bfloat16), one_g, one_g, b2_hi, b2_lo],
            axis=0)

        rm_acc = None
        col_mins = []
        for c in range(_NC):
            lo, hi = c * cb, (c + 1) * cb
            e = jax.lax.dot_general(
                p_aug, g_aug[:, lo:hi],
                (((1,), (0,)), ((), ())),
                preferred_element_type=jnp.float32,
            )
            for k in range(cb // 128):
                ek = e[:, k * 128:(k + 1) * 128]
                rm_acc = ek if rm_acc is None else jnp.minimum(rm_acc, ek)
            col_mins.append(jnp.min(e, axis=0, keepdims=True))

        row_min = jnp.min(rm_acc, axis=1, keepdims=True)
        s1 = s1 + jnp.sum(jnp.maximum(row_min, 0.0), keepdims=True)

        col_min = jnp.concatenate(col_mins, axis=1)
        s2 = s2 + jnp.sum(jnp.maximum(col_min, 0.0), keepdims=True)

    out_ref[...] = (s1 / (B * N)) + (s2 / (B * M))


def kernel(pred, gt):
    B, N, _ = pred.shape
    M = gt.shape[1]
    gs = -2.0 * jnp.swapaxes(gt, 1, 2)

    out = pl.pallas_call(
        _chamfer_tc_kernel,
        out_shape=jax.ShapeDtypeStruct((1, 1), jnp.float32),
    )(pred, gs)
    return out[0, 0]


# speedup vs baseline: 1.1218x; 1.0846x over previous
import jax
import jax.numpy as jnp
from jax.experimental import pallas as pl
from jax.experimental.pallas import tpu as pltpu

_NC = 8


def _chamfer_tc_kernel(pred_ref, gs_ref, out_ref, s1_acc, s2_acc):
    b = pl.program_id(0)
    nb = pl.num_programs(0)

    p = pred_ref[0]
    gs = gs_ref[0]
    N = p.shape[0]
    M = gs.shape[1]

    one_p = jnp.ones((N, 1), jnp.bfloat16)
    one_g = jnp.ones((1, M), jnp.bfloat16)

    a2 = jnp.sum(p * p, axis=1, keepdims=True)
    a2_hi = a2.astype(jnp.bfloat16)
    a2_lo = (a2 - a2_hi.astype(jnp.float32)).astype(jnp.bfloat16)
    b2 = 0.25 * jnp.sum(gs * gs, axis=0, keepdims=True)
    b2_hi = b2.astype(jnp.bfloat16)
    b2_lo = (b2 - b2_hi.astype(jnp.float32)).astype(jnp.bfloat16)

    p_aug = jnp.concatenate(
        [p.astype(jnp.bfloat16), a2_hi, a2_lo, one_p, one_p], axis=1)
    g_aug = jnp.concatenate(
        [gs.astype(jnp.bfloat16), one_g, one_g, b2_hi, b2_lo], axis=0)

    @pl.when(b == 0)
    def _init_sums():
        s1_acc[...] = jnp.zeros_like(s1_acc)
        s2_acc[...] = jnp.zeros_like(s2_acc)

    cb = M // _NC
    rm_acc = None
    col_mins = []
    for c in range(_NC):
        lo, hi = c * cb, (c + 1) * cb
        e = jax.lax.dot_general(
            p_aug, g_aug[:, lo:hi],
            (((1,), (0,)), ((), ())),
            preferred_element_type=jnp.float32,
        )
        for k in range(cb // 128):
            ek = e[:, k * 128:(k + 1) * 128]
            rm_acc = ek if rm_acc is None else jnp.minimum(rm_acc, ek)
        col_mins.append(jnp.min(e, axis=0, keepdims=True))

    row_min = jnp.min(rm_acc, axis=1, keepdims=True)
    s1_acc[...] += jnp.sum(jnp.maximum(row_min, 0.0), keepdims=True)

    col_min = jnp.concatenate(col_mins, axis=1)
    s2_acc[...] += jnp.sum(jnp.maximum(col_min, 0.0), keepdims=True)

    @pl.when(b == nb - 1)
    def _emit():
        out_ref[...] = (s1_acc[...] / (nb * N)) + (s2_acc[...] / (nb * M))


def kernel(pred, gt):
    B, N, _ = pred.shape
    M = gt.shape[1]
    gs = -2.0 * jnp.swapaxes(gt, 1, 2)

    out = pl.pallas_call(
        _chamfer_tc_kernel,
        grid=(B,),
        in_specs=[
            pl.BlockSpec((1, N, 3), lambda b: (b, 0, 0)),
            pl.BlockSpec((1, 3, M), lambda b: (b, 0, 0)),
        ],
        out_specs=pl.BlockSpec((1, 1), lambda b: (0, 0)),
        out_shape=jax.ShapeDtypeStruct((1, 1), jnp.float32),
        scratch_shapes=[
            pltpu.VMEM((1, 1), jnp.float32),
            pltpu.VMEM((1, 1), jnp.float32),
        ],
    )(pred, gs)
    return out[0, 0]
